# baseline (speedup 1.0000x reference)
import functools

import jax
import jax.numpy as jnp
from jax import lax
from jax.experimental import pallas as pl
from jax.experimental.pallas import tpu as pltpu
from jax.experimental.pallas import tpu_sc as plsc

N = 100000
E = 3200000
RAW = 128
R1, R2, R3 = 20, 15, 12
R5, R6, R7 = 10, 20, 10

D = 16
NC, NS = 2, 16
NW = NC * NS
CHUNK = 128
SUP = 4
NSUP = 196
CPW = SUP * NSUP
NCHUNKS = E // CHUNK
ROWS_PER_TILE = 6272
N_ACC = ROWS_PER_TILE * NS

BN = 4096
GRID = -(-N // BN)
PB = BN * D // 128
NP = N * D // 128

_MESH = plsc.VectorSubcoreMesh(core_axis_name="c", subcore_axis_name="s")


@functools.partial(
    pl.kernel,
    out_type=jax.ShapeDtypeStruct((NC, N_ACC, D), jnp.float32),
    mesh=_MESH,
    scratch_types=[
        pltpu.VMEM((3, SUP, CHUNK), jnp.int32),
        pltpu.VMEM((3, SUP, CHUNK), jnp.int32),
        pltpu.VMEM((3, SUP, CHUNK, D), jnp.float32),
        pltpu.VMEM_SHARED((N_ACC, D), jnp.float32),
        pltpu.SemaphoreType.DMA,
        pltpu.SemaphoreType.DMA,
        pltpu.SemaphoreType.DMA,
    ],
    compiler_params=pltpu.CompilerParams(use_tc_tiling_on_sc=False),
)
def _sc_agg(table_hbm, edges_hbm, zeros_hbm, out_hbm,
            src_v, dst_v, rows_v, acc_sh, gsem, ssem, isem):
    c = lax.axis_index("c")
    s = lax.axis_index("s")
    base = s * ROWS_PER_TILE

    pltpu.sync_copy(zeros_hbm, acc_sh.at[pl.ds(base, ROWS_PER_TILE)])

    plsc.subcore_barrier()

    wbase = (c * NS + s) * CPW
    ulim = jnp.minimum(NSUP, (NCHUNKS - wbase) // SUP)

    def _stage_idx(u, p):
        pltpu.async_copy(edges_hbm.at[0].at[pl.ds(wbase + u * SUP, SUP)],
                         src_v.at[p], isem)
        pltpu.async_copy(edges_hbm.at[1].at[pl.ds(wbase + u * SUP, SUP)],
                         dst_v.at[p], isem)

    def _drain_idx():
        for _ in range(2):
            pltpu.make_async_copy(edges_hbm.at[0].at[pl.ds(0, SUP)],
                                  src_v.at[0], isem).wait()

    def _fire_gathers(p):
        @pl.loop(0, SUP)
        def _(j):
            pltpu.async_copy(table_hbm.at[src_v.at[p, j]], rows_v.at[p, j], gsem)

    _stage_idx(0, 0)
    _drain_idx()
    _fire_gathers(0)
    _stage_idx(1, 1)
    _drain_idx()
    _fire_gathers(1)
    _stage_idx(2, 2)

    @pl.loop(0, ulim)
    def _(u):
        p = lax.rem(u, 3)

        @pl.loop(0, SUP)
        def _(j):
            pltpu.make_async_copy(table_hbm.at[pl.ds(0, CHUNK)],
                                  rows_v.at[p, 0], gsem).wait()

        @pl.when(u + 2 < ulim)
        def _():
            _drain_idx()
            _fire_gathers(lax.rem(u + 2, 3))

        @pl.loop(0, SUP)
        def _(j):
            pltpu.async_copy(rows_v.at[p, j], acc_sh.at[dst_v.at[p, j]],
                             ssem, add=True)

        @pl.loop(0, SUP)
        def _(j):
            pltpu.make_async_copy(rows_v.at[p, 0],
                                  acc_sh.at[pl.ds(0, CHUNK)], ssem).wait()

        @pl.when(u + 3 < ulim)
        def _():
            _stage_idx(u + 3, p)

    plsc.subcore_barrier()
    pltpu.sync_copy(acc_sh.at[pl.ds(base, ROWS_PER_TILE)],
                    out_hbm.at[c].at[pl.ds(base, ROWS_PER_TILE)])


def _full(shape):
    return pl.BlockSpec(shape, lambda i: tuple(0 for _ in shape))


def _rows(shape):
    ndims = len(shape)
    if ndims == 2:
        return pl.BlockSpec(shape, lambda i: (i, 0))
    return pl.BlockSpec(shape, lambda i: (0, i, 0))


def _unpack(a):
    return jnp.concatenate([a[:, g * D:(g + 1) * D] for g in range(8)], axis=0)


def _pack(a):
    return jnp.concatenate([a[g * PB:(g + 1) * PB] for g in range(8)], axis=1)


def _mlp_body(x_ref, w1_ref, b1_ref, n5_ref, w2_ref, b2_ref, n6_ref,
              w3_ref, b3_ref, n7_ref, t0_ref):
    xp = x_ref[...]
    xa = jnp.concatenate([xp[:, g:g + 1] for g in range(8)], axis=0)
    h = jnp.maximum(xa * w1_ref[...] + b1_ref[...], 0.0) + n5_ref[...]
    h = jnp.maximum(jnp.dot(h, w2_ref[...], preferred_element_type=jnp.float32)
                    + b2_ref[...], 0.0) + n6_ref[...]
    h = jnp.maximum(jnp.dot(h, w3_ref[...], preferred_element_type=jnp.float32)
                    + b3_ref[...], 0.0) + n7_ref[...]
    lane = lax.broadcasted_iota(jnp.int32, (BN, D), 1)
    t0_ref[...] = _pack(h[:, :D] + jnp.where(lane == R7, 1.0, 0.0))


def _l1root_body(t0_ref, wroot_ref, brel_ref, r1a_ref, r1b_ref):
    r = (jnp.dot(_unpack(t0_ref[...]), wroot_ref[...],
                 preferred_element_type=jnp.float32) + brel_ref[...])
    r1a_ref[...] = _pack(r[:, :D])
    r1b_ref[...] = _pack(r[:, D:2 * D])


def _l1comb_body(p_ref, r1a_ref, r1b_ref, wrel_ref, na_ref, wnext_ref,
                 t1_ref, h1a_ref, h1b_ref, invp_ref):
    pr = p_ref[...]
    sp = _unpack(pr[0] + pr[1])
    inv = 1.0 / jnp.maximum(sp[:, R7:R7 + 1], 1.0)
    r32 = jnp.concatenate([_unpack(r1a_ref[...]), _unpack(r1b_ref[...])], axis=1)
    h1 = jnp.maximum(
        jnp.dot(sp * inv, wrel_ref[...], preferred_element_type=jnp.float32)
        + r32, 0.0) + na_ref[...]
    t1_ref[...] = _pack(jnp.dot(h1, wnext_ref[...],
                                preferred_element_type=jnp.float32))
    h1a_ref[...] = _pack(h1[:, :D])
    h1b_ref[...] = _pack(h1[:, D:2 * D])
    invp_ref[...] = _pack(jnp.broadcast_to(inv, (BN, D)))


def _l2root_body(h1a_ref, h1b_ref, wroot_ref, brel_ref, r_ref):
    h1u = jnp.concatenate([_unpack(h1a_ref[...]), _unpack(h1b_ref[...])], axis=1)
    r_ref[...] = _pack(jnp.dot(h1u, wroot_ref[...],
                               preferred_element_type=jnp.float32)
                       + brel_ref[...])


def _root16_body(h_ref, wroot_ref, brel_ref, r_ref):
    r_ref[...] = _pack(jnp.dot(_unpack(h_ref[...]), wroot_ref[...],
                               preferred_element_type=jnp.float32)
                       + brel_ref[...])


def _comb2_body(p_ref, r_ref, invp_ref, nb_ref, wnext_ref, t2_ref, h2_ref):
    pr = p_ref[...]
    mean16 = _unpack((pr[0] + pr[1]) * invp_ref[...])
    h2 = jnp.maximum(mean16 + _unpack(r_ref[...]), 0.0) + nb_ref[...]
    t2_ref[...] = _pack(jnp.dot(h2, wnext_ref[...],
                                preferred_element_type=jnp.float32))
    h2_ref[...] = _pack(h2)


def _comb3_body(p_ref, r_ref, invp_ref, nc_ref, t3_ref):
    pr = p_ref[...]
    mean16 = _unpack((pr[0] + pr[1]) * invp_ref[...])
    t3_ref[...] = _pack(jnp.maximum(mean16 + _unpack(r_ref[...]), 0.0)
                        + nc_ref[...])


def _l4root_body(t3_ref, wroot_ref, brel_ref, r_ref):
    r_ref[...] = (jnp.dot(_unpack(t3_ref[...]), wroot_ref[...],
                          preferred_element_type=jnp.float32) + brel_ref[...])


def _comb4_body(p_ref, r_ref, invp_ref, wrel_ref, o_ref):
    pr = p_ref[...]
    mean16 = _unpack((pr[0] + pr[1]) * invp_ref[...])
    o = jnp.dot(mean16, wrel_ref[...],
                preferred_element_type=jnp.float32) + r_ref[...]
    o_ref[...] = jnp.concatenate(
        [o[g * PB:(g + 1) * PB][:, None, :] for g in range(8)], axis=1)


def _pad2(w, rin, rout):
    return jnp.zeros((rin, rout), jnp.float32).at[:w.shape[0], :w.shape[1]].set(w)


def _pad1(v, r=128):
    return jnp.zeros((1, r), jnp.float32).at[0, :v.shape[0]].set(v)


def _noise_vals():
    nk = jax.random.split(jax.random.key(42), 6)
    return (jax.random.normal(nk[0], (R5,), dtype=jnp.float32),
            jax.random.normal(nk[1], (R6,), dtype=jnp.float32),
            jax.random.normal(nk[2], (R7,), dtype=jnp.float32),
            jax.random.normal(nk[3], (R1,), dtype=jnp.float32),
            jax.random.normal(nk[4], (R2,), dtype=jnp.float32),
            jax.random.normal(nk[5], (R3,), dtype=jnp.float32))


_PK = pl.BlockSpec((PB, 128), lambda i: (i, 0))


def kernel(x, edge_index, W1, b1, W2, b2, W3, b3,
           Wrel1, brel1, Wroot1, Wrel2, brel2, Wroot2,
           Wrel3, brel3, Wroot3, Wrel4, brel4, Wroot4):
    n5, n6, n7, na, nb, nc = _noise_vals()
    f32 = jnp.float32
    pk = jax.ShapeDtypeStruct((NP, 128), f32)

    w1 = _pad1(W1[0])
    w2 = _pad2(W2, 128, 128)
    w3 = _pad2(W3, 128, 128)
    wrel1 = _pad2(Wrel1, D, 32)
    wroot1 = _pad2(Wroot1, D, 32)
    wnext1 = _pad2(Wrel2, 32, D)
    wroot2 = _pad2(Wroot2, 32, D)
    wnext2 = _pad2(Wrel3, D, D)
    wroot3 = _pad2(Wroot3, D, D)
    wrel4 = _pad2(Wrel4, D, 128)
    wroot4 = _pad2(Wroot4, D, 128)

    zrows = jnp.zeros((ROWS_PER_TILE, D), f32)

    e3 = edge_index.reshape(2, NCHUNKS, CHUNK)
    xp = x.reshape(NP, 8)

    t0 = pl.pallas_call(
        _mlp_body,
        grid=(GRID,),
        in_specs=[pl.BlockSpec((PB, 8), lambda i: (i, 0)), _full((1, 128)),
                  _full((1, 128)), _full((1, 128)), _full((128, 128)),
                  _full((1, 128)), _full((1, 128)), _full((128, 128)),
                  _full((1, 128)), _full((1, 128))],
        out_specs=_PK,
        out_shape=pk,
    )(xp, w1, _pad1(b1), _pad1(n5), w2, _pad1(b2), _pad1(n6), w3, _pad1(b3), _pad1(n7))

    r1a, r1b = pl.pallas_call(
        _l1root_body,
        grid=(GRID,),
        in_specs=[_PK, _full((D, 32)), _full((1, 32))],
        out_specs=[_PK, _PK],
        out_shape=[pk, pk],
    )(t0, wroot1, _pad1(brel1, 32))

    p1 = _sc_agg(t0.reshape(N, D), e3, zrows).reshape(NC, N_ACC // 8, 128)

    t1, h1a, h1b, invp = pl.pallas_call(
        _l1comb_body,
        grid=(GRID,),
        in_specs=[pl.BlockSpec((2, PB, 128), lambda i: (0, i, 0)), _PK, _PK,
                  _full((D, 32)), _full((1, 32)), _full((32, D))],
        out_specs=[_PK] * 4,
        out_shape=[pk] * 4,
    )(p1, r1a, r1b, wrel1, _pad1(na, 32), wnext1)

    r2 = pl.pallas_call(
        _l2root_body,
        grid=(GRID,),
        in_specs=[_PK, _PK, _full((32, D)), _full((1, D))],
        out_specs=_PK,
        out_shape=pk,
    )(h1a, h1b, wroot2, _pad1(brel2, D))

    p2 = _sc_agg(t1.reshape(N, D), e3, zrows).reshape(NC, N_ACC // 8, 128)

    t2, h2 = pl.pallas_call(
        _comb2_body,
        grid=(GRID,),
        in_specs=[pl.BlockSpec((2, PB, 128), lambda i: (0, i, 0)), _PK, _PK,
                  _full((1, D)), _full((D, D))],
        out_specs=[_PK] * 2,
        out_shape=[pk] * 2,
    )(p2, r2, invp, _pad1(nb, D), wnext2)

    r3 = pl.pallas_call(
        _root16_body,
        grid=(GRID,),
        in_specs=[_PK, _full((D, D)), _full((1, D))],
        out_specs=_PK,
        out_shape=pk,
    )(h2, wroot3, _pad1(brel3, D))

    p3 = _sc_agg(t2.reshape(N, D), e3, zrows).reshape(NC, N_ACC // 8, 128)

    t3 = pl.pallas_call(
        _comb3_body,
        grid=(GRID,),
        in_specs=[pl.BlockSpec((2, PB, 128), lambda i: (0, i, 0)), _PK, _PK,
                  _full((1, D))],
        out_specs=_PK,
        out_shape=pk,
    )(p3, r3, invp, _pad1(nc, D))

    r4 = pl.pallas_call(
        _l4root_body,
        grid=(GRID,),
        in_specs=[_PK, _full((D, 128)), _full((1, 128))],
        out_specs=pl.BlockSpec((BN, 128), lambda i: (i, 0)),
        out_shape=jax.ShapeDtypeStruct((GRID * BN, 128), f32),
    )(t3, wroot4, _pad1(brel4))

    p4 = _sc_agg(t3.reshape(N, D), e3, zrows).reshape(NC, N_ACC // 8, 128)

    out = pl.pallas_call(
        _comb4_body,
        grid=(GRID,),
        in_specs=[pl.BlockSpec((2, PB, 128), lambda i: (0, i, 0)),
                  pl.BlockSpec((BN, 128), lambda i: (i, 0)), _PK,
                  _full((D, 128))],
        out_specs=pl.BlockSpec((PB, 8, 128), lambda i: (i, 0, 0)),
        out_shape=jax.ShapeDtypeStruct((NP, 8, 128), f32),
    )(p4, r4, invp, wrel4)

    return out.reshape(N, 128)


# speedup vs baseline: 1.0074x; 1.0074x over previous
import functools

import jax
import jax.numpy as jnp
from jax import lax
from jax.experimental import pallas as pl
from jax.experimental.pallas import tpu as pltpu
from jax.experimental.pallas import tpu_sc as plsc

N = 100000
E = 3200000
RAW = 128
R1, R2, R3 = 20, 15, 12
R5, R6, R7 = 10, 20, 10

D = 16
NC, NS = 2, 16
NW = NC * NS
CHUNK = 128
SUP = 4
NSUP = 196
CPW = SUP * NSUP
NCHUNKS = E // CHUNK
ROWS_PER_TILE = 6272
N_ACC = ROWS_PER_TILE * NS

BN = 4096
GRID = -(-N // BN)
PB = BN * D // 128
NP = N * D // 128

_MESH = plsc.VectorSubcoreMesh(core_axis_name="c", subcore_axis_name="s")


@functools.partial(
    pl.kernel,
    out_type=jax.ShapeDtypeStruct((NC, N_ACC, D), jnp.float32),
    mesh=_MESH,
    scratch_types=[
        pltpu.VMEM((3, SUP, CHUNK), jnp.int32),
        pltpu.VMEM((3, SUP, CHUNK), jnp.int32),
        pltpu.VMEM((3, SUP, CHUNK, D), jnp.float32),
        pltpu.VMEM_SHARED((N_ACC, D), jnp.float32),
        pltpu.SemaphoreType.DMA,
        pltpu.SemaphoreType.DMA,
        pltpu.SemaphoreType.DMA,
    ],
    compiler_params=pltpu.CompilerParams(use_tc_tiling_on_sc=False),
)
def _sc_agg(table_hbm, edges_hbm, zeros_hbm, out_hbm,
            src_v, dst_v, rows_v, acc_sh, gsem, ssem, isem):
    c = lax.axis_index("c")
    s = lax.axis_index("s")
    base = s * ROWS_PER_TILE

    pltpu.sync_copy(zeros_hbm, acc_sh.at[pl.ds(base, ROWS_PER_TILE)])

    plsc.subcore_barrier()

    wbase = (c * NS + s) * CPW
    ulim = jnp.minimum(NSUP, (NCHUNKS - wbase) // SUP)

    def _stage_idx(u, p):
        pltpu.async_copy(edges_hbm.at[0].at[pl.ds(wbase + u * SUP, SUP)],
                         src_v.at[p], isem)
        pltpu.async_copy(edges_hbm.at[1].at[pl.ds(wbase + u * SUP, SUP)],
                         dst_v.at[p], isem)

    def _drain_idx():
        for _ in range(2):
            pltpu.make_async_copy(edges_hbm.at[0].at[pl.ds(0, SUP)],
                                  src_v.at[0], isem).wait()

    def _fire_gathers(p):
        @pl.loop(0, SUP)
        def _(j):
            pltpu.async_copy(table_hbm.at[src_v.at[p, j]], rows_v.at[p, j], gsem)

    _stage_idx(0, 0)
    _drain_idx()
    _fire_gathers(0)
    _stage_idx(1, 1)
    _drain_idx()
    _fire_gathers(1)
    _stage_idx(2, 2)

    @pl.loop(0, ulim)
    def _(u):
        p = lax.rem(u, 3)

        @pl.loop(0, SUP)
        def _(j):
            pltpu.make_async_copy(table_hbm.at[pl.ds(0, CHUNK)],
                                  rows_v.at[p, 0], gsem).wait()

        @pl.when(u + 2 < ulim)
        def _():
            _drain_idx()
            _fire_gathers(lax.rem(u + 2, 3))

        @pl.loop(0, SUP)
        def _(j):
            pltpu.async_copy(rows_v.at[p, j], acc_sh.at[dst_v.at[p, j]],
                             ssem, add=True)

        @pl.loop(0, SUP)
        def _(j):
            pltpu.make_async_copy(rows_v.at[p, 0],
                                  acc_sh.at[pl.ds(0, CHUNK)], ssem).wait()

        @pl.when(u + 3 < ulim)
        def _():
            _stage_idx(u + 3, p)

    plsc.subcore_barrier()
    pltpu.sync_copy(acc_sh.at[pl.ds(base, ROWS_PER_TILE)],
                    out_hbm.at[c].at[pl.ds(base, ROWS_PER_TILE)])


def _full(shape):
    return pl.BlockSpec(shape, lambda i: tuple(0 for _ in shape))


def _rows(shape):
    ndims = len(shape)
    if ndims == 2:
        return pl.BlockSpec(shape, lambda i: (i, 0))
    return pl.BlockSpec(shape, lambda i: (0, i, 0))


def _unpack(a):
    return jnp.concatenate([a[:, g * D:(g + 1) * D] for g in range(8)], axis=0)


def _pack(a):
    return jnp.concatenate([a[g * PB:(g + 1) * PB] for g in range(8)], axis=1)


def _mlp_body(x_ref, w1_ref, b1_ref, n5_ref, w2_ref, b2_ref, n6_ref,
              w3_ref, b3_ref, n7_ref, t0_ref):
    xp = x_ref[...]
    xa = jnp.concatenate([xp[:, g:g + 1] for g in range(8)], axis=0)
    h = jnp.maximum(xa * w1_ref[...] + b1_ref[...], 0.0) + n5_ref[...]
    h = jnp.maximum(jnp.dot(h, w2_ref[...], preferred_element_type=jnp.float32)
                    + b2_ref[...], 0.0) + n6_ref[...]
    h = jnp.maximum(jnp.dot(h, w3_ref[...], preferred_element_type=jnp.float32)
                    + b3_ref[...], 0.0) + n7_ref[...]
    lane = lax.broadcasted_iota(jnp.int32, (BN, D), 1)
    t0_ref[...] = _pack(h[:, :D] + jnp.where(lane == R7, 1.0, 0.0))


def _l1_body(p_ref, t0_ref, wrel_ref, brel_ref, wroot_ref, na_ref, wnext_ref,
             t1_ref, h1a_ref, h1b_ref, invp_ref):
    pr = p_ref[...]
    sp = _unpack(pr[0] + pr[1])
    inv = 1.0 / jnp.maximum(sp[:, R7:R7 + 1], 1.0)
    t0u = _unpack(t0_ref[...])
    h1 = jnp.maximum(
        jnp.dot(sp * inv, wrel_ref[...], preferred_element_type=jnp.float32)
        + brel_ref[...]
        + jnp.dot(t0u, wroot_ref[...], preferred_element_type=jnp.float32),
        0.0) + na_ref[...]
    t1_ref[...] = _pack(jnp.dot(h1, wnext_ref[...],
                                preferred_element_type=jnp.float32))
    h1a_ref[...] = _pack(h1[:, :D])
    h1b_ref[...] = _pack(h1[:, D:2 * D])
    invp_ref[...] = _pack(jnp.broadcast_to(inv, (BN, D)))


def _l2_body(p_ref, h1a_ref, h1b_ref, invp_ref, brel_ref, wroot_ref, nb_ref,
             wnext_ref, t2_ref, h2_ref):
    pr = p_ref[...]
    mean16 = _unpack((pr[0] + pr[1]) * invp_ref[...])
    h1u = jnp.concatenate([_unpack(h1a_ref[...]), _unpack(h1b_ref[...])], axis=1)
    h2 = jnp.maximum(
        mean16 + brel_ref[...]
        + jnp.dot(h1u, wroot_ref[...], preferred_element_type=jnp.float32),
        0.0) + nb_ref[...]
    t2_ref[...] = _pack(jnp.dot(h2, wnext_ref[...],
                                preferred_element_type=jnp.float32))
    h2_ref[...] = _pack(h2)


def _l3_body(p_ref, h2_ref, invp_ref, brel_ref, wroot_ref, nc_ref, t3_ref):
    pr = p_ref[...]
    mean16 = _unpack((pr[0] + pr[1]) * invp_ref[...])
    h3 = jnp.maximum(
        mean16 + brel_ref[...]
        + jnp.dot(_unpack(h2_ref[...]), wroot_ref[...],
                  preferred_element_type=jnp.float32),
        0.0) + nc_ref[...]
    t3_ref[...] = _pack(h3)


def _l4_body(p_ref, t3_ref, invp_ref, wrel_ref, brel_ref, wroot_ref, o_ref):
    pr = p_ref[...]
    mean16 = _unpack((pr[0] + pr[1]) * invp_ref[...])
    o = (jnp.dot(mean16, wrel_ref[...], preferred_element_type=jnp.float32)
         + brel_ref[...]
         + jnp.dot(_unpack(t3_ref[...]), wroot_ref[...],
                   preferred_element_type=jnp.float32))
    o_ref[...] = jnp.concatenate(
        [o[g * PB:(g + 1) * PB][:, None, :] for g in range(8)], axis=1)


def _pad2(w, rin, rout):
    return jnp.zeros((rin, rout), jnp.float32).at[:w.shape[0], :w.shape[1]].set(w)


def _pad1(v, r=128):
    return jnp.zeros((1, r), jnp.float32).at[0, :v.shape[0]].set(v)


def _noise_vals():
    nk = jax.random.split(jax.random.key(42), 6)
    return (jax.random.normal(nk[0], (R5,), dtype=jnp.float32),
            jax.random.normal(nk[1], (R6,), dtype=jnp.float32),
            jax.random.normal(nk[2], (R7,), dtype=jnp.float32),
            jax.random.normal(nk[3], (R1,), dtype=jnp.float32),
            jax.random.normal(nk[4], (R2,), dtype=jnp.float32),
            jax.random.normal(nk[5], (R3,), dtype=jnp.float32))


_PK = pl.BlockSpec((PB, 128), lambda i: (i, 0))


def kernel(x, edge_index, W1, b1, W2, b2, W3, b3,
           Wrel1, brel1, Wroot1, Wrel2, brel2, Wroot2,
           Wrel3, brel3, Wroot3, Wrel4, brel4, Wroot4):
    n5, n6, n7, na, nb, nc = _noise_vals()
    f32 = jnp.float32
    pk = jax.ShapeDtypeStruct((NP, 128), f32)
    pspec = pl.BlockSpec((2, PB, 128), lambda i: (0, i, 0))

    w1 = _pad1(W1[0])
    w2 = _pad2(W2, 128, 128)
    w3 = _pad2(W3, 128, 128)
    wrel1 = _pad2(Wrel1, D, 128)
    wroot1 = _pad2(Wroot1, D, 128)
    wnext1 = _pad2(Wrel2, 128, D)
    wroot2 = _pad2(Wroot2, 32, D)
    wnext2 = _pad2(Wrel3, D, D)
    wroot3 = _pad2(Wroot3, D, D)
    wrel4 = _pad2(Wrel4, D, 128)
    wroot4 = _pad2(Wroot4, D, 128)

    zrows = jnp.zeros((ROWS_PER_TILE, D), f32)

    e3 = edge_index.reshape(2, NCHUNKS, CHUNK)
    xp = x.reshape(NP, 8)

    t0 = pl.pallas_call(
        _mlp_body,
        grid=(GRID,),
        in_specs=[pl.BlockSpec((PB, 8), lambda i: (i, 0)), _full((1, 128)),
                  _full((1, 128)), _full((1, 128)), _full((128, 128)),
                  _full((1, 128)), _full((1, 128)), _full((128, 128)),
                  _full((1, 128)), _full((1, 128))],
        out_specs=_PK,
        out_shape=pk,
    )(xp, w1, _pad1(b1), _pad1(n5), w2, _pad1(b2), _pad1(n6), w3, _pad1(b3), _pad1(n7))

    p1 = _sc_agg(t0.reshape(N, D), e3, zrows).reshape(NC, N_ACC // 8, 128)

    t1, h1a, h1b, invp = pl.pallas_call(
        _l1_body,
        grid=(GRID,),
        in_specs=[pspec, _PK, _full((D, 128)), _full((1, 128)),
                  _full((D, 128)), _full((1, 128)), _full((128, D))],
        out_specs=[_PK] * 4,
        out_shape=[pk] * 4,
    )(p1, t0, wrel1, _pad1(brel1), wroot1, _pad1(na), wnext1)

    p2 = _sc_agg(t1.reshape(N, D), e3, zrows).reshape(NC, N_ACC // 8, 128)

    t2, h2 = pl.pallas_call(
        _l2_body,
        grid=(GRID,),
        in_specs=[pspec, _PK, _PK, _PK, _full((1, D)), _full((32, D)),
                  _full((1, D)), _full((D, D))],
        out_specs=[_PK] * 2,
        out_shape=[pk] * 2,
    )(p2, h1a, h1b, invp, _pad1(brel2, D), wroot2, _pad1(nb, D), wnext2)

    p3 = _sc_agg(t2.reshape(N, D), e3, zrows).reshape(NC, N_ACC // 8, 128)

    t3 = pl.pallas_call(
        _l3_body,
        grid=(GRID,),
        in_specs=[pspec, _PK, _PK, _full((1, D)), _full((D, D)), _full((1, D))],
        out_specs=_PK,
        out_shape=pk,
    )(p3, h2, invp, _pad1(brel3, D), wroot3, _pad1(nc, D))

    p4 = _sc_agg(t3.reshape(N, D), e3, zrows).reshape(NC, N_ACC // 8, 128)

    out = pl.pallas_call(
        _l4_body,
        grid=(GRID,),
        in_specs=[pspec, _PK, _PK, _full((D, 128)), _full((1, 128)),
                  _full((D, 128))],
        out_specs=pl.BlockSpec((PB, 8, 128), lambda i: (i, 0, 0)),
        out_shape=jax.ShapeDtypeStruct((NP, 8, 128), f32),
    )(p4, t3, invp, wrel4, _pad1(brel4), wroot4)

    return out.reshape(N, 128)
